# baseline (speedup 1.0000x reference)
import functools

import jax
import jax.numpy as jnp
from jax import lax
from jax.experimental import pallas as pl
from jax.experimental.pallas import tpu as pltpu
from jax.experimental.pallas import tpu_sc as plsc

B = 16384
NUM_CORES = 2
NUM_SUBCORES = 16
NUM_WORKERS = NUM_CORES * NUM_SUBCORES
BPW = B // NUM_WORKERS
D = 32
NU = 1000000
NA = 100000
BN = 65536
NU_PAD = 16 * BN
BNA = 7168
NA_PAD = 14 * BNA
INV_B = 1.0 / B


def _mv_body(w_ref, xu1_ref, xu2_ref, xa_ref, ou_ref, oa_ref):
    w = w_ref[...]
    wu1 = w[:, 0:D // 2]
    wu2 = w[:, D // 2:D]
    wa = w[:, D:2 * D]
    dn = (((1,), (0,)), ((), ()))
    o1 = jax.lax.dot_general(
        wu1, xu1_ref[...], dn, preferred_element_type=jnp.float32)
    o2 = jax.lax.dot_general(
        wu2, xu2_ref[...], dn, preferred_element_type=jnp.float32)
    ou_ref[...] = (o1 + o2)[0]
    oa_ref[...] = jax.lax.dot_general(
        wa, xa_ref[...], dn, preferred_element_type=jnp.float32)[0]


def _matvec2(ut, at, wb):
    return pl.pallas_call(
        _mv_body,
        grid=(16,),
        in_specs=[
            pl.BlockSpec((1, 128), lambda i: (0, 0)),
            pl.BlockSpec((D // 2, BN), lambda i: (0, i)),
            pl.BlockSpec((D // 2, BN), lambda i: (1, i)),
            pl.BlockSpec((D, BNA), lambda i: (0, jnp.minimum(i, 13))),
        ],
        out_specs=(
            pl.BlockSpec((BN,), lambda i: (i,)),
            pl.BlockSpec((BNA,), lambda i: (jnp.minimum(i, 13),)),
        ),
        out_shape=(
            jax.ShapeDtypeStruct((NU_PAD,), jnp.float32),
            jax.ShapeDtypeStruct((NA_PAD,), jnp.float32),
        ),
    )(wb, ut, ut, at)


_mesh = plsc.VectorSubcoreMesh(core_axis_name="c", subcore_axis_name="s")


def _sc_body(pu_ref, pa_ref, users_ref, articles_ref, ratings_ref, wb_ref,
             out_ref, lp_ref,
             uidx_v, aidx_v, rat_v, puv, pav, out_v, wb_v, part_v, all_v,
             shared_v, usem, asem):
    c = lax.axis_index("c")
    s = lax.axis_index("s")
    wid = s * NUM_CORES + c
    base = pl.multiple_of(wid * BPW, BPW)

    ins = [
        pltpu.async_copy(users_ref.at[pl.ds(base, BPW)], uidx_v, usem),
        pltpu.async_copy(articles_ref.at[pl.ds(base, BPW)], aidx_v, asem),
        pltpu.async_copy(ratings_ref.at[pl.ds(base, BPW)], rat_v, usem),
        pltpu.async_copy(wb_ref, wb_v, asem),
    ]
    for cp in ins:
        cp.wait()

    cps = []
    for j in range(4):
        cps.append(pltpu.async_copy(pu_ref.at[uidx_v.at[pl.ds(j * 128, 128)]],
                                    puv.at[pl.ds(j * 128, 128)], usem))
        cps.append(pltpu.async_copy(pa_ref.at[aidx_v.at[pl.ds(j * 128, 128)]],
                                    pav.at[pl.ds(j * 128, 128)], asem))
    for cp in cps:
        cp.wait()

    bias = wb_v[0, pl.ds(64, 16)][0]
    lane = lax.iota(jnp.int32, 16)

    def group(g, lacc):
        off = pl.multiple_of(g * 16, 16)
        acc = puv[pl.ds(off, 16)] + pav[pl.ds(off, 16)] + bias
        out_v[pl.ds(off, 16)] = acc
        diff = acc - rat_v[pl.ds(off, 16)]
        return lacc + diff * diff

    lacc = lax.fori_loop(0, BPW // 16, group, jnp.zeros((16,), jnp.float32))

    pltpu.sync_copy(out_v, out_ref.at[pl.ds(base, BPW)])

    part_v[pl.ds(0, 16)] = lacc * INV_B
    pltpu.sync_copy(part_v, shared_v.at[s])
    plsc.subcore_barrier()

    @pl.when(s == 0)
    def _():
        pltpu.sync_copy(shared_v, all_v)

        def fold(i, acc):
            return acc + all_v[i, pl.ds(0, 16)]

        acc = lax.fori_loop(1, NUM_SUBCORES, fold, all_v[0, pl.ds(0, 16)])
        total = jnp.sum(acc)
        part_v[pl.ds(0, 16)] = jnp.where(
            lane == 0, jnp.full((16,), total, jnp.float32),
            jnp.zeros((16,), jnp.float32))
        pltpu.sync_copy(part_v, lp_ref.at[c])


_sc_call = pl.kernel(
    _sc_body,
    out_type=(
        jax.ShapeDtypeStruct((B,), jnp.float32),
        jax.ShapeDtypeStruct((NUM_CORES, 128), jnp.float32),
    ),
    mesh=_mesh,
    compiler_params=pltpu.CompilerParams(needs_layout_passes=False),
    scratch_types=[
        pltpu.VMEM((BPW,), jnp.int32),
        pltpu.VMEM((BPW,), jnp.int32),
        pltpu.VMEM((BPW,), jnp.float32),
        pltpu.VMEM((BPW,), jnp.float32),
        pltpu.VMEM((BPW,), jnp.float32),
        pltpu.VMEM((BPW,), jnp.float32),
        pltpu.VMEM((1, 128), jnp.float32),
        pltpu.VMEM((128,), jnp.float32),
        pltpu.VMEM((16, 128), jnp.float32),
        pltpu.VMEM_SHARED((16, 128), jnp.float32),
        pltpu.SemaphoreType.DMA,
        pltpu.SemaphoreType.DMA,
    ],
)


def kernel(users, articles, ratings, user_table, article_table, fc_w, fc_b):
    users_r = users.astype(jnp.int32)
    articles_r = articles.astype(jnp.int32)
    wb = jnp.concatenate([fc_w.reshape(-1), fc_b.reshape(-1),
                          jnp.zeros(128 - 2 * D - 1, jnp.float32)])
    wb2 = wb.reshape(1, 128)
    pu, pa = _matvec2(user_table.T, article_table.T, wb2)
    out_r, lp = _sc_call(pu, pa, users_r, articles_r, ratings, wb2)
    output = out_r.reshape(B, 1)
    loss = lp[0, 0] + lp[1, 0]
    return (output, loss)


# speedup vs baseline: 1.0063x; 1.0063x over previous
import functools

import jax
import jax.numpy as jnp
from jax import lax
from jax.experimental import pallas as pl
from jax.experimental.pallas import tpu as pltpu
from jax.experimental.pallas import tpu_sc as plsc

B = 16384
NUM_CORES = 2
NUM_SUBCORES = 16
NUM_WORKERS = NUM_CORES * NUM_SUBCORES
BPW = B // NUM_WORKERS
D = 32
NU = 1000000
NA = 100000
BN = 65536
NU_PAD = 16 * BN
BNA = 7168
NA_PAD = 14 * BNA
INV_B = 1.0 / B


def _mv_body(w_ref, xu_ref, xa_ref, ou_ref, oa_ref):
    w = w_ref[...]
    wu = w[:, 0:D]
    wa = w[:, D:2 * D]
    dn = (((1,), (0,)), ((), ()))
    ou_ref[...] = jax.lax.dot_general(
        wu, xu_ref[...], dn, preferred_element_type=jnp.float32)[0]
    oa_ref[...] = jax.lax.dot_general(
        wa, xa_ref[...], dn, preferred_element_type=jnp.float32)[0]


def _matvec2(ut, at, wb):
    return pl.pallas_call(
        _mv_body,
        grid=(16,),
        in_specs=[
            pl.BlockSpec((1, 128), lambda i: (0, 0)),
            pl.BlockSpec((D, BN), lambda i: (0, i)),
            pl.BlockSpec((D, BNA), lambda i: (0, jnp.minimum(i, 13))),
        ],
        out_specs=(
            pl.BlockSpec((BN,), lambda i: (i,)),
            pl.BlockSpec((BNA,), lambda i: (jnp.minimum(i, 13),)),
        ),
        out_shape=(
            jax.ShapeDtypeStruct((NU_PAD,), jnp.float32),
            jax.ShapeDtypeStruct((NA_PAD,), jnp.float32),
        ),
    )(wb, ut, at)


_mesh = plsc.VectorSubcoreMesh(core_axis_name="c", subcore_axis_name="s")


def _sc_body(pu_ref, pa_ref, users_ref, articles_ref, ratings_ref, wb_ref,
             out_ref, lp_ref,
             uidx_v, aidx_v, rat_v, puv, pav, out_v, wb_v, part_v, all_v,
             shared_v, usem, asem):
    c = lax.axis_index("c")
    s = lax.axis_index("s")
    wid = s * NUM_CORES + c
    base = pl.multiple_of(wid * BPW, BPW)

    ins = [
        pltpu.async_copy(users_ref.at[pl.ds(base, BPW)], uidx_v, usem),
        pltpu.async_copy(articles_ref.at[pl.ds(base, BPW)], aidx_v, asem),
        pltpu.async_copy(ratings_ref.at[pl.ds(base, BPW)], rat_v, usem),
        pltpu.async_copy(wb_ref, wb_v, asem),
    ]
    for cp in ins:
        cp.wait()

    cps = []
    for j in range(4):
        cps.append(pltpu.async_copy(pu_ref.at[uidx_v.at[pl.ds(j * 128, 128)]],
                                    puv.at[pl.ds(j * 128, 128)], usem))
        cps.append(pltpu.async_copy(pa_ref.at[aidx_v.at[pl.ds(j * 128, 128)]],
                                    pav.at[pl.ds(j * 128, 128)], asem))
    for cp in cps:
        cp.wait()

    bias = wb_v[0, pl.ds(64, 16)][0]
    lane = lax.iota(jnp.int32, 16)

    def group(g, lacc):
        off = pl.multiple_of(g * 16, 16)
        acc = puv[pl.ds(off, 16)] + pav[pl.ds(off, 16)] + bias
        out_v[pl.ds(off, 16)] = acc
        diff = acc - rat_v[pl.ds(off, 16)]
        return lacc + diff * diff

    lacc = lax.fori_loop(0, BPW // 16, group, jnp.zeros((16,), jnp.float32))

    pltpu.sync_copy(out_v, out_ref.at[pl.ds(base, BPW)])

    part_v[pl.ds(0, 16)] = lacc * INV_B
    pltpu.sync_copy(part_v, shared_v.at[s])
    plsc.subcore_barrier()

    @pl.when(s == 0)
    def _():
        pltpu.sync_copy(shared_v, all_v)

        def fold(i, acc):
            return acc + all_v[i, pl.ds(0, 16)]

        acc = lax.fori_loop(1, NUM_SUBCORES, fold, all_v[0, pl.ds(0, 16)])
        total = jnp.sum(acc)
        part_v[pl.ds(0, 16)] = jnp.where(
            lane == 0, jnp.full((16,), total, jnp.float32),
            jnp.zeros((16,), jnp.float32))
        pltpu.sync_copy(part_v, lp_ref.at[c])


_sc_call = pl.kernel(
    _sc_body,
    out_type=(
        jax.ShapeDtypeStruct((B,), jnp.float32),
        jax.ShapeDtypeStruct((NUM_CORES, 128), jnp.float32),
    ),
    mesh=_mesh,
    compiler_params=pltpu.CompilerParams(needs_layout_passes=False),
    scratch_types=[
        pltpu.VMEM((BPW,), jnp.int32),
        pltpu.VMEM((BPW,), jnp.int32),
        pltpu.VMEM((BPW,), jnp.float32),
        pltpu.VMEM((BPW,), jnp.float32),
        pltpu.VMEM((BPW,), jnp.float32),
        pltpu.VMEM((BPW,), jnp.float32),
        pltpu.VMEM((1, 128), jnp.float32),
        pltpu.VMEM((128,), jnp.float32),
        pltpu.VMEM((16, 128), jnp.float32),
        pltpu.VMEM_SHARED((16, 128), jnp.float32),
        pltpu.SemaphoreType.DMA,
        pltpu.SemaphoreType.DMA,
    ],
)


def kernel(users, articles, ratings, user_table, article_table, fc_w, fc_b):
    users_r = users.astype(jnp.int32)
    articles_r = articles.astype(jnp.int32)
    wb = jnp.concatenate([fc_w.reshape(-1), fc_b.reshape(-1),
                          jnp.zeros(128 - 2 * D - 1, jnp.float32)])
    wb2 = wb.reshape(1, 128)
    pu, pa = _matvec2(user_table.T, article_table.T, wb2)
    out_r, lp = _sc_call(pu, pa, users_r, articles_r, ratings, wb2)
    output = out_r.reshape(B, 1)
    loss = lp[0, 0] + lp[1, 0]
    return (output, loss)


# speedup vs baseline: 1.0752x; 1.0685x over previous
import functools

import jax
import jax.numpy as jnp
from jax import lax
from jax.experimental import pallas as pl
from jax.experimental.pallas import tpu as pltpu
from jax.experimental.pallas import tpu_sc as plsc

B = 16384
NUM_CORES = 1
NUM_SUBCORES = 16
NUM_WORKERS = NUM_CORES * NUM_SUBCORES
BPW = B // NUM_WORKERS
D = 32
NU = 1000000
NA = 100000
BN = 65536
NU_PAD = 16 * BN
BNA = 7168
NA_PAD = 14 * BNA
INV_B = 1.0 / B


def _mv_body(w_ref, xu_ref, xa_ref, ou_ref, oa_ref):
    w = w_ref[...]
    wu = w[:, 0:D]
    wa = w[:, D:2 * D]
    dn = (((1,), (0,)), ((), ()))
    ou_ref[...] = jax.lax.dot_general(
        wu, xu_ref[...], dn, preferred_element_type=jnp.float32)[0]
    oa_ref[...] = jax.lax.dot_general(
        wa, xa_ref[...], dn, preferred_element_type=jnp.float32)[0]


def _matvec2(ut, at, wb):
    return pl.pallas_call(
        _mv_body,
        grid=(16,),
        in_specs=[
            pl.BlockSpec((1, 128), lambda i: (0, 0)),
            pl.BlockSpec((D, BN), lambda i: (0, i)),
            pl.BlockSpec((D, BNA), lambda i: (0, jnp.minimum(i, 13))),
        ],
        out_specs=(
            pl.BlockSpec((BN,), lambda i: (i,)),
            pl.BlockSpec((BNA,), lambda i: (jnp.minimum(i, 13),)),
        ),
        out_shape=(
            jax.ShapeDtypeStruct((NU_PAD,), jnp.float32),
            jax.ShapeDtypeStruct((NA_PAD,), jnp.float32),
        ),
    )(wb, ut, at)


_mesh = plsc.VectorSubcoreMesh(core_axis_name="c", subcore_axis_name="s",
                               num_cores=NUM_CORES)


def _sc_body(pu_ref, pa_ref, users_ref, articles_ref, ratings_ref, wb_ref,
             out_ref, lp_ref,
             uidx_v, aidx_v, rat_v, puv, pav, out_v, wb_v, part_v, all_v,
             shared_v, usem, asem):
    c = lax.axis_index("c")
    s = lax.axis_index("s")
    wid = s * NUM_CORES + c
    base = pl.multiple_of(wid * BPW, BPW)

    ins = [
        pltpu.async_copy(users_ref.at[pl.ds(base, BPW)], uidx_v, usem),
        pltpu.async_copy(articles_ref.at[pl.ds(base, BPW)], aidx_v, asem),
        pltpu.async_copy(ratings_ref.at[pl.ds(base, BPW)], rat_v, usem),
        pltpu.async_copy(wb_ref, wb_v, asem),
    ]
    for cp in ins:
        cp.wait()

    cps = []
    for j in range(BPW // 128):
        cps.append(pltpu.async_copy(pu_ref.at[uidx_v.at[pl.ds(j * 128, 128)]],
                                    puv.at[pl.ds(j * 128, 128)], usem))
        cps.append(pltpu.async_copy(pa_ref.at[aidx_v.at[pl.ds(j * 128, 128)]],
                                    pav.at[pl.ds(j * 128, 128)], asem))
    for cp in cps:
        cp.wait()

    bias = wb_v[0, pl.ds(64, 16)][0]
    lane = lax.iota(jnp.int32, 16)

    def group(g, lacc):
        off = pl.multiple_of(g * 16, 16)
        acc = puv[pl.ds(off, 16)] + pav[pl.ds(off, 16)] + bias
        out_v[pl.ds(off, 16)] = acc
        diff = acc - rat_v[pl.ds(off, 16)]
        return lacc + diff * diff

    lacc = lax.fori_loop(0, BPW // 16, group, jnp.zeros((16,), jnp.float32))

    pltpu.sync_copy(out_v, out_ref.at[pl.ds(base, BPW)])

    part_v[pl.ds(0, 16)] = lacc * INV_B
    pltpu.sync_copy(part_v, shared_v.at[s])
    plsc.subcore_barrier()

    @pl.when(s == 0)
    def _():
        pltpu.sync_copy(shared_v, all_v)

        def fold(i, acc):
            return acc + all_v[i, pl.ds(0, 16)]

        acc = lax.fori_loop(1, NUM_SUBCORES, fold, all_v[0, pl.ds(0, 16)])
        total = jnp.sum(acc)
        part_v[pl.ds(0, 16)] = jnp.where(
            lane == 0, jnp.full((16,), total, jnp.float32),
            jnp.zeros((16,), jnp.float32))
        pltpu.sync_copy(part_v, lp_ref.at[c])


_sc_call = pl.kernel(
    _sc_body,
    out_type=(
        jax.ShapeDtypeStruct((B,), jnp.float32),
        jax.ShapeDtypeStruct((NUM_CORES, 128), jnp.float32),
    ),
    mesh=_mesh,
    compiler_params=pltpu.CompilerParams(needs_layout_passes=False),
    scratch_types=[
        pltpu.VMEM((BPW,), jnp.int32),
        pltpu.VMEM((BPW,), jnp.int32),
        pltpu.VMEM((BPW,), jnp.float32),
        pltpu.VMEM((BPW,), jnp.float32),
        pltpu.VMEM((BPW,), jnp.float32),
        pltpu.VMEM((BPW,), jnp.float32),
        pltpu.VMEM((1, 128), jnp.float32),
        pltpu.VMEM((128,), jnp.float32),
        pltpu.VMEM((16, 128), jnp.float32),
        pltpu.VMEM_SHARED((16, 128), jnp.float32),
        pltpu.SemaphoreType.DMA,
        pltpu.SemaphoreType.DMA,
    ],
)


def kernel(users, articles, ratings, user_table, article_table, fc_w, fc_b):
    users_r = users.astype(jnp.int32)
    articles_r = articles.astype(jnp.int32)
    wb = jnp.concatenate([fc_w.reshape(-1), fc_b.reshape(-1),
                          jnp.zeros(128 - 2 * D - 1, jnp.float32)])
    wb2 = wb.reshape(1, 128)
    pu, pa = _matvec2(user_table.T, article_table.T, wb2)
    out_r, lp = _sc_call(pu, pa, users_r, articles_r, ratings, wb2)
    output = out_r.reshape(B, 1)
    loss = lp[0, 0] if NUM_CORES == 1 else lp[0, 0] + lp[1, 0]
    return (output, loss)
